# Initial kernel scaffold; baseline (speedup 1.0000x reference)
import jax
import jax.numpy as jnp
from jax.experimental import pallas as pl

_NS = 8


def _stats_body(x_ref, rowmax_ref, rowcnt_ref, gmax_ref, f_ref):
    xb = x_ref[...]
    rowmax = jnp.max(xb, axis=2)
    rowcnt = jnp.sum((xb == rowmax[:, :, None]).astype(jnp.float32), axis=2)
    rowmax_ref[...] = rowmax
    rowcnt_ref[...] = rowcnt
    gmax_ref[...] = jnp.max(rowmax, axis=1).reshape(1, 1, -1)
    f_ref[...] = (jnp.sum(jnp.sum(xb, axis=2), axis=1) / xb.shape[2]).reshape(1, 1, -1)


def _finish_body(rowmax_ref, rowcnt_ref, gmax_ref, f_ref, out_ref):
    B, C, H = rowmax_ref.shape
    rm = rowmax_ref[...]
    rc = rowcnt_ref[...]
    gm = gmax_ref[...]
    row = jnp.sum(jnp.where(rm == gm[:, :, None], rc, 0.0), axis=1)
    csum = jnp.cumsum(row, axis=1)
    hexc = csum - row
    prev_j = jnp.zeros((B, 1), jnp.int32)
    ok = jnp.ones((B, 1), jnp.bool_)
    hks = [jnp.zeros((B, 1), jnp.float32)]
    for k in range(1, _NS):
        thr = jnp.floor(jnp.float32(k) * C / _NS)
        jk = jnp.sum((hexc <= thr).astype(jnp.int32), axis=1, keepdims=True) - 1
        good = (jk >= 1) & (jk <= H - 2) & (jk > prev_j)
        ok = ok & good
        hks.append(jnp.where(ok, jk.astype(jnp.float32), 0.0))
        prev_j = jk
    hks.append(jnp.full((B, 1), float(H), jnp.float32))
    hk = jnp.concatenate(hks, axis=1)
    d = hk[:, 1:] - hk[:, :-1]
    f = f_ref[...]
    out_ref[...] = f[:, None, :] / d[:, :, None]


def kernel(x):
    B, C, H, W = x.shape
    x3 = x.reshape(B * C, H, W)
    CB = 8
    rowmax, rowcnt, gmax3, f3 = pl.pallas_call(
        _stats_body,
        grid=(B * C // CB,),
        in_specs=[pl.BlockSpec((CB, H, W), lambda g: (g, 0, 0))],
        out_specs=[
            pl.BlockSpec((CB, H), lambda g: (g, 0)),
            pl.BlockSpec((CB, H), lambda g: (g, 0)),
            pl.BlockSpec((1, 1, CB), lambda g: (g, 0, 0)),
            pl.BlockSpec((1, 1, CB), lambda g: (g, 0, 0)),
        ],
        out_shape=[
            jax.ShapeDtypeStruct((B * C, H), jnp.float32),
            jax.ShapeDtypeStruct((B * C, H), jnp.float32),
            jax.ShapeDtypeStruct((B * C // CB, 1, CB), jnp.float32),
            jax.ShapeDtypeStruct((B * C // CB, 1, CB), jnp.float32),
        ],
    )(x3)
    out3 = pl.pallas_call(
        _finish_body,
        out_shape=jax.ShapeDtypeStruct((B, _NS, C), jnp.float32),
    )(
        rowmax.reshape(B, C, H),
        rowcnt.reshape(B, C, H),
        gmax3.reshape(B, C),
        f3.reshape(B, C),
    )
    return out3.reshape(B, _NS * C)


# speedup vs baseline: 2.9740x; 2.9740x over previous
import jax
import jax.numpy as jnp
from jax.experimental import pallas as pl

_NS = 8


def _stats_body(x_ref, rowmax_ref, rowcnt_ref, gmax_ref, f_ref):
    xb = x_ref[...]
    rowmax = jnp.max(xb, axis=2)
    rowcnt = jnp.sum((xb == rowmax[:, :, None]).astype(jnp.float32), axis=2)
    rowmax_ref[...] = rowmax
    rowcnt_ref[...] = rowcnt
    gmax_ref[...] = jnp.max(rowmax, axis=1).reshape(1, 1, -1)
    f_ref[...] = (jnp.sum(jnp.sum(xb, axis=2), axis=1) / xb.shape[2]).reshape(1, 1, -1)


def _finish_body(rowmax_ref, rowcnt_ref, gmax_ref, f_ref, out_ref):
    B, C, H = rowmax_ref.shape
    rm = rowmax_ref[...]
    rc = rowcnt_ref[...]
    gm = gmax_ref[...]
    row = jnp.sum(jnp.where(rm == gm[:, :, None], rc, 0.0), axis=1)
    csum = row
    sh = 1
    while sh < H:
        csum = csum + jnp.concatenate(
            [jnp.zeros((B, sh), jnp.float32), csum[:, : H - sh]], axis=1
        )
        sh *= 2
    hexc = csum - row
    prev_j = jnp.zeros((B, 1), jnp.int32)
    ok = jnp.ones((B, 1), jnp.bool_)
    hks = [jnp.zeros((B, 1), jnp.float32)]
    for k in range(1, _NS):
        thr = jnp.floor(jnp.float32(k) * C / _NS)
        jk = jnp.sum((hexc <= thr).astype(jnp.int32), axis=1, keepdims=True) - 1
        good = (jk >= 1) & (jk <= H - 2) & (jk > prev_j)
        ok = ok & good
        hks.append(jnp.where(ok, jk.astype(jnp.float32), 0.0))
        prev_j = jk
    hks.append(jnp.full((B, 1), float(H), jnp.float32))
    hk = jnp.concatenate(hks, axis=1)
    d = hk[:, 1:] - hk[:, :-1]
    f = f_ref[...]
    out_ref[...] = f[:, None, :] / d[:, :, None]


def kernel(x):
    B, C, H, W = x.shape
    x3 = x.reshape(B * C, H, W)
    CB = 8
    rowmax, rowcnt, gmax3, f3 = pl.pallas_call(
        _stats_body,
        grid=(B * C // CB,),
        in_specs=[pl.BlockSpec((CB, H, W), lambda g: (g, 0, 0))],
        out_specs=[
            pl.BlockSpec((CB, H), lambda g: (g, 0)),
            pl.BlockSpec((CB, H), lambda g: (g, 0)),
            pl.BlockSpec((1, 1, CB), lambda g: (g, 0, 0)),
            pl.BlockSpec((1, 1, CB), lambda g: (g, 0, 0)),
        ],
        out_shape=[
            jax.ShapeDtypeStruct((B * C, H), jnp.float32),
            jax.ShapeDtypeStruct((B * C, H), jnp.float32),
            jax.ShapeDtypeStruct((B * C // CB, 1, CB), jnp.float32),
            jax.ShapeDtypeStruct((B * C // CB, 1, CB), jnp.float32),
        ],
    )(x3)
    out3 = pl.pallas_call(
        _finish_body,
        out_shape=jax.ShapeDtypeStruct((B, _NS, C), jnp.float32),
    )(
        rowmax.reshape(B, C, H),
        rowcnt.reshape(B, C, H),
        gmax3.reshape(B, C),
        f3.reshape(B, C),
    )
    return out3.reshape(B, _NS * C)
